# baseline (speedup 1.0000x reference)
import jax
import jax.numpy as jnp
from jax import lax
from jax.experimental import pallas as pl
from jax.experimental.pallas import tpu as pltpu
from jax.experimental.pallas import tpu_sc as plsc

_N = 10000
_D = 128
_E = 320000
_NC = 2
_NS = 16
_NT = _NC * _NS
_EPAD = 327680
_EPT = _EPAD // _NT
_K = 80
_NCHUNK = _EPT // _K
_W = 32
_NW = _NCHUNK // _W
_NP = 10240
_RPT = _NP // _NS


def _hop_body(g_hbm, src_hbm, dst_hbm, out_hbm, sw0, sw1, dw0, dw1, rows0,
              rows1, acc, sg0, sg1, si0, si1, di0, di1):
    cid = lax.axis_index("c")
    sid = lax.axis_index("s")
    wid = cid * _NS + sid

    swin = (sw0, sw1)
    dwin = (dw0, dw1)
    rows = (rows0, rows1)
    gsem = (sg0, sg1)
    ssem = (si0, si1)
    dsem = (di0, di1)

    def start_win(w, p):
        pltpu.async_copy(src_hbm.at[wid, pl.ds(w * _W, _W)], swin[p], ssem[p])
        pltpu.async_copy(dst_hbm.at[wid, pl.ds(w * _W, _W)], dwin[p], dsem[p])

    def wait_win(p):
        pltpu.make_async_copy(src_hbm.at[0, pl.ds(0, _W)], swin[p],
                              ssem[p]).wait()
        pltpu.make_async_copy(dst_hbm.at[0, pl.ds(0, _W)], dwin[p],
                              dsem[p]).wait()

    def start_gather(b, p, c):
        pltpu.async_copy(g_hbm.at[swin[p].at[c]], rows[b], gsem[b])

    def wait_gather(b):
        pltpu.make_async_copy(g_hbm.at[swin[0].at[0]], rows[b],
                              gsem[b]).wait()

    def scatter(b, p, c):
        pltpu.sync_copy(rows[b], acc.at[dwin[p].at[c]], add=True)

    def fill_zeros(r, _):
        for c8 in range(_D // 16):
            rows0[r, pl.ds(c8 * 16, 16)] = jnp.zeros((16,), jnp.float32)
        return 0

    lax.fori_loop(0, _K, fill_zeros, 0)
    for z in range(_RPT // _K):
        pltpu.sync_copy(rows0, acc.at[pl.ds(sid * _RPT + z * _K, _K)])

    start_win(0, 0)
    plsc.subcore_barrier()

    for w in range(_NW):
        p = w % 2
        wait_win(p)
        if w + 1 < _NW:
            start_win(w + 1, 1 - p)
        start_gather(0, p, 0)
        start_gather(1, p, 1)

        def pair(j, _, p=p):
            c0 = 2 * j
            wait_gather(0)
            scatter(0, p, c0)
            start_gather(0, p, c0 + 2)
            wait_gather(1)
            scatter(1, p, c0 + 1)
            start_gather(1, p, c0 + 3)
            return 0

        lax.fori_loop(0, _W // 2 - 1, pair, 0)
        wait_gather(0)
        scatter(0, p, _W - 2)
        wait_gather(1)
        scatter(1, p, _W - 1)

    plsc.subcore_barrier()
    pltpu.sync_copy(acc.at[pl.ds(sid * _RPT, _RPT)],
                    out_hbm.at[cid, pl.ds(sid * _RPT, _RPT)])


def _sc_hop(g, src3, dst3):
    mesh = plsc.VectorSubcoreMesh(core_axis_name="c", subcore_axis_name="s",
                                  num_cores=_NC, num_subcores=_NS)
    return pl.kernel(
        _hop_body,
        out_type=jax.ShapeDtypeStruct((_NC, _NP, _D), jnp.float32),
        mesh=mesh,
        scratch_types=[
            pltpu.VMEM((_W, _K), jnp.int32),
            pltpu.VMEM((_W, _K), jnp.int32),
            pltpu.VMEM((_W, _K), jnp.int32),
            pltpu.VMEM((_W, _K), jnp.int32),
            pltpu.VMEM((_K, _D), jnp.float32),
            pltpu.VMEM((_K, _D), jnp.float32),
            pltpu.VMEM_SHARED((_NP, _D), jnp.float32),
            pltpu.SemaphoreType.DMA,
            pltpu.SemaphoreType.DMA,
            pltpu.SemaphoreType.DMA,
            pltpu.SemaphoreType.DMA,
            pltpu.SemaphoreType.DMA,
            pltpu.SemaphoreType.DMA,
        ],
    )(g, src3, dst3)


_BLK = 1000


def _tc_transform_body(feat_ref, w_ref, b_ref, degp_ref, out_ref):
    dp = degp_ref[...]
    deg = dp[0, :, 0] + dp[1, :, 0]
    nrm = lax.rsqrt(jnp.maximum(deg, 1.0))
    h = jnp.dot(feat_ref[...], w_ref[...],
                preferred_element_type=jnp.float32) + b_ref[...]
    out_ref[...] = h * nrm[:, None]


def _tc_transform(feat, w, b2, degp):
    return pl.pallas_call(
        _tc_transform_body,
        grid=(_N // _BLK,),
        in_specs=[
            pl.BlockSpec((_BLK, _D), lambda i: (i, 0)),
            pl.BlockSpec((_D, _D), lambda i: (0, 0)),
            pl.BlockSpec((1, _D), lambda i: (0, 0)),
            pl.BlockSpec((_NC, _BLK, _D), lambda i: (0, i, 0)),
        ],
        out_specs=pl.BlockSpec((_BLK, _D), lambda i: (i, 0)),
        out_shape=jax.ShapeDtypeStruct((_N, _D), jnp.float32),
    )(feat, w, b2, degp)


def _tc_scale_body(s_ref, degp_ref, out_ref, power):
    dp = degp_ref[...]
    deg = dp[0, :, 0] + dp[1, :, 0]
    nrm = lax.rsqrt(jnp.maximum(deg, 1.0))
    scale = nrm * nrm if power == 2 else nrm
    s = s_ref[0] + s_ref[1]
    out_ref[...] = s * scale[:, None]


def _tc_scale(s, degp, power):
    body = lambda a, b, o: _tc_scale_body(a, b, o, power)
    return pl.pallas_call(
        body,
        grid=(_N // _BLK,),
        in_specs=[
            pl.BlockSpec((_NC, _BLK, _D), lambda i: (0, i, 0)),
            pl.BlockSpec((_NC, _BLK, _D), lambda i: (0, i, 0)),
        ],
        out_specs=pl.BlockSpec((_BLK, _D), lambda i: (i, 0)),
        out_shape=jax.ShapeDtypeStruct((_N, _D), jnp.float32),
    )(s, degp)


def kernel(feat, edge_index, W, b):
    pad = _EPAD - _E
    src_p = jnp.concatenate(
        [edge_index[0], jnp.zeros((pad,), jnp.int32)])
    dst_p = jnp.concatenate(
        [edge_index[1], jnp.full((pad,), _N, jnp.int32)])
    src3 = src_p.reshape(_NT, _NCHUNK, _K)
    dst3 = dst_p.reshape(_NT, _NCHUNK, _K)
    b2 = b.reshape(1, _D)
    ones = jnp.ones((_N, _D), jnp.float32)
    degp = _sc_hop(ones, src3, dst3)
    g0 = _tc_transform(feat, W, b2, degp)
    s1 = _sc_hop(g0, src3, dst3)
    g1 = _tc_scale(s1, degp, 2)
    s2 = _sc_hop(g1, src3, dst3)
    return _tc_scale(s2, degp, 1)


# speedup vs baseline: 1.4939x; 1.4939x over previous
import jax
import jax.numpy as jnp
from jax import lax
from jax.experimental import pallas as pl
from jax.experimental.pallas import tpu as pltpu
from jax.experimental.pallas import tpu_sc as plsc

_N = 10000
_D = 128
_E = 320000
_NC = 2
_NS = 16
_NT = _NC * _NS
_EPAD = 327680
_EPT = _EPAD // _NT
_K = 80
_NCHUNK = _EPT // _K
_W = 32
_NW = _NCHUNK // _W
_NP = 10240
_RPT = _NP // _NS


def _hop_body(g_hbm, src_hbm, dst_hbm, out_hbm, sw0, sw1, dw0, dw1, rows0,
              rows1, acc, sg0, sg1, si0, si1, di0, di1):
    cid = lax.axis_index("c")
    sid = lax.axis_index("s")
    wid = cid * _NS + sid

    swin = (sw0, sw1)
    dwin = (dw0, dw1)
    rows = (rows0, rows1)
    gsem = (sg0, sg1)
    ssem = (si0, si1)
    dsem = (di0, di1)

    def start_win(w, p):
        pltpu.async_copy(src_hbm.at[wid, pl.ds(w * _W, _W)], swin[p], ssem[p])
        pltpu.async_copy(dst_hbm.at[wid, pl.ds(w * _W, _W)], dwin[p], dsem[p])

    def wait_win(p):
        pltpu.make_async_copy(src_hbm.at[0, pl.ds(0, _W)], swin[p],
                              ssem[p]).wait()
        pltpu.make_async_copy(dst_hbm.at[0, pl.ds(0, _W)], dwin[p],
                              dsem[p]).wait()

    def start_gather(b, p, c):
        pltpu.async_copy(g_hbm.at[swin[p].at[c]], rows[b], gsem[b])

    def wait_gather(b):
        pltpu.make_async_copy(g_hbm.at[swin[0].at[0]], rows[b],
                              gsem[b]).wait()

    def scatter(b, p, c):
        pltpu.sync_copy(rows[b], acc.at[dwin[p].at[c]], add=True)

    def fill_zeros(r, _):
        for c8 in range(_D // 16):
            rows0[r, pl.ds(c8 * 16, 16)] = jnp.zeros((16,), jnp.float32)
        return 0

    lax.fori_loop(0, _K, fill_zeros, 0)
    for z in range(_RPT // _K):
        pltpu.sync_copy(rows0, acc.at[pl.ds(sid * _RPT + z * _K, _K)])

    start_win(0, 0)
    plsc.subcore_barrier()

    for w in range(_NW):
        p = w % 2
        wait_win(p)
        if w + 1 < _NW:
            start_win(w + 1, 1 - p)
        start_gather(0, p, 0)
        start_gather(1, p, 1)

        def pair(j, _, p=p):
            c0 = 2 * j
            wait_gather(0)
            scatter(0, p, c0)
            start_gather(0, p, c0 + 2)
            wait_gather(1)
            scatter(1, p, c0 + 1)
            start_gather(1, p, c0 + 3)
            return 0

        lax.fori_loop(0, _W // 2 - 1, pair, 0)
        wait_gather(0)
        scatter(0, p, _W - 2)
        wait_gather(1)
        scatter(1, p, _W - 1)

    plsc.subcore_barrier()
    pltpu.sync_copy(acc.at[pl.ds(sid * _RPT, _RPT)],
                    out_hbm.at[cid, pl.ds(sid * _RPT, _RPT)])


def _deg_body(dst_hbm, out_hbm, dw0, dw1, ones_v, zero_v, acc, di0, di1):
    cid = lax.axis_index("c")
    sid = lax.axis_index("s")
    wid = cid * _NS + sid

    dwin = (dw0, dw1)
    dsem = (di0, di1)

    def start_win(w, p):
        pltpu.async_copy(dst_hbm.at[wid, pl.ds(w * _W, _W)], dwin[p], dsem[p])

    def wait_win(p):
        pltpu.make_async_copy(dst_hbm.at[0, pl.ds(0, _W)], dwin[p],
                              dsem[p]).wait()

    def fill(r, _):
        for c8 in range(_D // 16):
            ones_v[r, pl.ds(c8 * 16, 16)] = jnp.ones((16,), jnp.float32)
            zero_v[r, pl.ds(c8 * 16, 16)] = jnp.zeros((16,), jnp.float32)
        return 0

    lax.fori_loop(0, _K, fill, 0)
    for z in range(_RPT // _K):
        pltpu.sync_copy(zero_v, acc.at[pl.ds(sid * _RPT + z * _K, _K)])

    start_win(0, 0)
    plsc.subcore_barrier()

    for w in range(_NW):
        p = w % 2
        wait_win(p)
        if w + 1 < _NW:
            start_win(w + 1, 1 - p)

        def chunk(c, _, p=p):
            pltpu.sync_copy(ones_v, acc.at[dwin[p].at[c]], add=True)
            return 0

        lax.fori_loop(0, _W, chunk, 0)

    plsc.subcore_barrier()
    pltpu.sync_copy(acc.at[pl.ds(sid * _RPT, _RPT)],
                    out_hbm.at[cid, pl.ds(sid * _RPT, _RPT)])


def _sc_deg(dst3):
    mesh = plsc.VectorSubcoreMesh(core_axis_name="c", subcore_axis_name="s",
                                  num_cores=_NC, num_subcores=_NS)
    return pl.kernel(
        _deg_body,
        out_type=jax.ShapeDtypeStruct((_NC, _NP, _D), jnp.float32),
        mesh=mesh,
        scratch_types=[
            pltpu.VMEM((_W, _K), jnp.int32),
            pltpu.VMEM((_W, _K), jnp.int32),
            pltpu.VMEM((_K, _D), jnp.float32),
            pltpu.VMEM((_K, _D), jnp.float32),
            pltpu.VMEM_SHARED((_NP, _D), jnp.float32),
            pltpu.SemaphoreType.DMA,
            pltpu.SemaphoreType.DMA,
        ],
    )(dst3)


def _sc_hop(g, src3, dst3):
    mesh = plsc.VectorSubcoreMesh(core_axis_name="c", subcore_axis_name="s",
                                  num_cores=_NC, num_subcores=_NS)
    return pl.kernel(
        _hop_body,
        out_type=jax.ShapeDtypeStruct((_NC, _NP, _D), jnp.float32),
        mesh=mesh,
        scratch_types=[
            pltpu.VMEM((_W, _K), jnp.int32),
            pltpu.VMEM((_W, _K), jnp.int32),
            pltpu.VMEM((_W, _K), jnp.int32),
            pltpu.VMEM((_W, _K), jnp.int32),
            pltpu.VMEM((_K, _D), jnp.float32),
            pltpu.VMEM((_K, _D), jnp.float32),
            pltpu.VMEM_SHARED((_NP, _D), jnp.float32),
            pltpu.SemaphoreType.DMA,
            pltpu.SemaphoreType.DMA,
            pltpu.SemaphoreType.DMA,
            pltpu.SemaphoreType.DMA,
            pltpu.SemaphoreType.DMA,
            pltpu.SemaphoreType.DMA,
        ],
    )(g, src3, dst3)


_BLK = 1000


def _tc_matmul_body(feat_ref, w_ref, b_ref, out_ref):
    out_ref[...] = jnp.dot(feat_ref[...], w_ref[...],
                           preferred_element_type=jnp.float32) + b_ref[...]


def _tc_matmul(feat, w, b2):
    return pl.pallas_call(
        _tc_matmul_body,
        grid=(_N // _BLK,),
        in_specs=[
            pl.BlockSpec((_BLK, _D), lambda i: (i, 0)),
            pl.BlockSpec((_D, _D), lambda i: (0, 0)),
            pl.BlockSpec((1, _D), lambda i: (0, 0)),
        ],
        out_specs=pl.BlockSpec((_BLK, _D), lambda i: (i, 0)),
        out_shape=jax.ShapeDtypeStruct((_N, _D), jnp.float32),
    )(feat, w, b2)


def _tc_scale_h_body(h_ref, degp_ref, out_ref):
    dp = degp_ref[...]
    deg = dp[0, :, 0] + dp[1, :, 0]
    nrm = lax.rsqrt(jnp.maximum(deg, 1.0))
    out_ref[...] = h_ref[...] * nrm[:, None]


def _tc_scale_h(h, degp):
    return pl.pallas_call(
        _tc_scale_h_body,
        grid=(_N // _BLK,),
        in_specs=[
            pl.BlockSpec((_BLK, _D), lambda i: (i, 0)),
            pl.BlockSpec((_NC, _BLK, _D), lambda i: (0, i, 0)),
        ],
        out_specs=pl.BlockSpec((_BLK, _D), lambda i: (i, 0)),
        out_shape=jax.ShapeDtypeStruct((_N, _D), jnp.float32),
    )(h, degp)


def _tc_scale_body(s_ref, degp_ref, out_ref, power):
    dp = degp_ref[...]
    deg = dp[0, :, 0] + dp[1, :, 0]
    nrm = lax.rsqrt(jnp.maximum(deg, 1.0))
    scale = nrm * nrm if power == 2 else nrm
    s = s_ref[0] + s_ref[1]
    out_ref[...] = s * scale[:, None]


def _tc_scale(s, degp, power):
    body = lambda a, b, o: _tc_scale_body(a, b, o, power)
    return pl.pallas_call(
        body,
        grid=(_N // _BLK,),
        in_specs=[
            pl.BlockSpec((_NC, _BLK, _D), lambda i: (0, i, 0)),
            pl.BlockSpec((_NC, _BLK, _D), lambda i: (0, i, 0)),
        ],
        out_specs=pl.BlockSpec((_BLK, _D), lambda i: (i, 0)),
        out_shape=jax.ShapeDtypeStruct((_N, _D), jnp.float32),
    )(s, degp)


def kernel(feat, edge_index, W, b):
    pad = _EPAD - _E
    src_p = jnp.concatenate(
        [edge_index[0], jnp.zeros((pad,), jnp.int32)])
    dst_p = jnp.concatenate(
        [edge_index[1], jnp.full((pad,), _N, jnp.int32)])
    src3 = src_p.reshape(_NT, _NCHUNK, _K)
    dst3 = dst_p.reshape(_NT, _NCHUNK, _K)
    b2 = b.reshape(1, _D)
    degp = _sc_deg(dst3)
    h = _tc_matmul(feat, W, b2)
    g0 = _tc_scale_h(h, degp)
    s1 = _sc_hop(g0, src3, dst3)
    g1 = _tc_scale(s1, degp, 2)
    s2 = _sc_hop(g1, src3, dst3)
    return _tc_scale(s2, degp, 1)
